# Initial kernel scaffold; baseline (speedup 1.0000x reference)
import jax
import jax.numpy as jnp
from jax.experimental import pallas as pl


def kernel(x, pe):
    raise NotImplementedError("write your pallas kernel here")


# speedup vs baseline: 1.4471x; 1.4471x over previous
import functools

import jax
import jax.numpy as jnp
from jax import lax
from jax.experimental import pallas as pl
from jax.experimental.pallas import tpu as pltpu
from jax.experimental.pallas import tpu_sc as plsc

HIDDEN = 128
B_TOTAL = 512 * 4 * 16 * 16
CHUNK = 128


def _make_gather():
    info = plsc.get_sparse_core_info()
    nc, ns = info.num_cores, info.num_subcores
    nw = nc * ns
    b_per_w = B_TOTAL // nw
    n_chunks = b_per_w // CHUNK
    mesh = plsc.VectorSubcoreMesh(core_axis_name="c", subcore_axis_name="s")

    @functools.partial(
        pl.kernel,
        mesh=mesh,
        out_type=jax.ShapeDtypeStruct((B_TOTAL, HIDDEN), jnp.float32),
        scratch_types=[
            pltpu.VMEM((CHUNK,), jnp.int32),
            pltpu.VMEM((CHUNK, HIDDEN), jnp.float32),
            pltpu.SemaphoreType.DMA,
        ],
    )
    def gather_kernel(x_hbm, pe_hbm, out_hbm, idx_v, rows_v, sem):
        wid = lax.axis_index("s") * nc + lax.axis_index("c")
        base = wid * b_per_w

        def body(i, carry):
            off = base + i * CHUNK
            pltpu.sync_copy(x_hbm.at[pl.ds(off, CHUNK)], idx_v)
            pltpu.async_copy(pe_hbm.at[idx_v], rows_v, sem).wait()
            pltpu.sync_copy(rows_v, out_hbm.at[pl.ds(off, CHUNK)])
            return carry

        lax.fori_loop(0, n_chunks, body, 0)

    return gather_kernel


def kernel(x, pe):
    orig_shape = x.shape
    flat = x.reshape(B_TOTAL).astype(jnp.int32)
    out = _make_gather()(flat, pe)
    return out.reshape(*orig_shape, HIDDEN)


# speedup vs baseline: 1.4544x; 1.0051x over previous
import functools

import jax
import jax.numpy as jnp
from jax import lax
from jax.experimental import pallas as pl
from jax.experimental.pallas import tpu as pltpu
from jax.experimental.pallas import tpu_sc as plsc

HIDDEN = 128
B_TOTAL = 512 * 4 * 16 * 16
CHUNK = 128


def _make_gather():
    info = plsc.get_sparse_core_info()
    nc, ns = info.num_cores, info.num_subcores
    nw = nc * ns
    b_per_w = B_TOTAL // nw
    n_chunks = b_per_w // CHUNK
    mesh = plsc.VectorSubcoreMesh(core_axis_name="c", subcore_axis_name="s")

    @functools.partial(
        pl.kernel,
        mesh=mesh,
        out_type=jax.ShapeDtypeStruct((B_TOTAL, HIDDEN), jnp.float32),
        scratch_types=[
            pltpu.VMEM((b_per_w,), jnp.int32),
            pltpu.VMEM((2, CHUNK, HIDDEN), jnp.float32),
            pltpu.SemaphoreType.DMA,
            pltpu.SemaphoreType.DMA,
        ],
    )
    def gather_kernel(x_hbm, pe_hbm, out_hbm, idx_v, rows_v, sem_g, sem_o):
        wid = lax.axis_index("s") * nc + lax.axis_index("c")
        base = wid * b_per_w
        pltpu.sync_copy(x_hbm.at[pl.ds(base, b_per_w)], idx_v)

        def g_copy(s, slot):
            return pltpu.make_async_copy(
                pe_hbm.at[idx_v.at[pl.ds(s * CHUNK, CHUNK)]],
                rows_v.at[slot], sem_g)

        def o_copy(s, slot):
            return pltpu.make_async_copy(
                rows_v.at[slot],
                out_hbm.at[pl.ds(base + s * CHUNK, CHUNK)], sem_o)

        def step(s, cur):
            oth = 1 - cur
            g_copy(s, cur).wait()
            o_copy(s, cur).start()
            o_copy(s - 1, oth).wait()
            g_copy(s + 1, oth).start()

        g_copy(0, 0).start()
        g_copy(0, 0).wait()
        o_copy(0, 0).start()
        g_copy(1, 1).start()

        def body(j, carry):
            s1 = 1 + 2 * j
            step(s1, 1)
            step(s1 + 1, 0)
            return carry

        lax.fori_loop(0, (n_chunks - 2) // 2, body, 0)

        g_copy(n_chunks - 1, 1).wait()
        o_copy(n_chunks - 1, 1).start()
        o_copy(n_chunks - 2, 0).wait()
        o_copy(n_chunks - 1, 1).wait()

    return gather_kernel


def kernel(x, pe):
    orig_shape = x.shape
    flat = x.reshape(B_TOTAL).astype(jnp.int32)
    out = _make_gather()(flat, pe)
    return out.reshape(*orig_shape, HIDDEN)
